# Initial kernel scaffold; baseline (speedup 1.0000x reference)
import jax
import jax.numpy as jnp
from jax.experimental import pallas as pl


def kernel(x, edge_index, batch, params):
    raise NotImplementedError("write your pallas kernel here")


# speedup vs baseline: 8.9332x; 8.9332x over previous
import functools

import jax
import jax.numpy as jnp
from jax import lax
from jax.experimental import pallas as pl
from jax.experimental.pallas import tpu as pltpu
from jax.experimental.pallas import tpu_sc as plsc

NC = 2
NS = 16
NW = NC * NS
K = 128
DR = 240
BM = 1000
_BN_INV = (1.0 + 1e-5) ** -0.5


def _sc_degree(dst_pad, n, ep):
    npad = n + DR
    ew = ep // NW
    nch = ew // K
    span = npad // NS
    mesh = plsc.VectorSubcoreMesh(core_axis_name="c", subcore_axis_name="s")

    @functools.partial(
        pl.kernel,
        out_type=jax.ShapeDtypeStruct((NC * npad,), jnp.float32),
        mesh=mesh,
        scratch_types=[
            pltpu.VMEM((K,), jnp.int32),
            pltpu.VMEM((K,), jnp.float32),
            pltpu.VMEM_SHARED((npad,), jnp.float32),
            pltpu.VMEM((npad // NS,), jnp.float32),
        ],
    )
    def deg_kernel(dst_hbm, out_hbm, dst_v, ones_v, acc, stage_v):
        c = lax.axis_index("c")
        s = lax.axis_index("s")
        wid = c * NS + s
        for i in range(K // 16):
            ones_v[pl.ds(i * 16, 16)] = jnp.full((16,), 1.0, jnp.float32)
        for i in range(span // 16):
            stage_v[pl.ds(i * 16, 16)] = jnp.zeros((16,), jnp.float32)
        pltpu.sync_copy(stage_v, acc.at[pl.ds(s * span, span)])
        plsc.subcore_barrier()
        base0 = wid * ew

        def chunk(j, carry):
            pltpu.sync_copy(dst_hbm.at[pl.ds(base0 + j * K, K)], dst_v)
            pltpu.sync_copy(ones_v, acc.at[dst_v], add=True)
            return carry

        lax.fori_loop(0, nch, chunk, 0)
        plsc.subcore_barrier()
        pltpu.sync_copy(acc.at[pl.ds(s * span, span)], stage_v)
        pltpu.sync_copy(stage_v, out_hbm.at[pl.ds(c * npad + s * span, span)])

    return deg_kernel(dst_pad).reshape(NC, npad)


def _sc_message(gs_pad, src_pad, dst_pad, n, ep):
    npad = n + DR
    ew = ep // NW
    nch = ew // K
    sp = npad // NS
    mesh = plsc.VectorSubcoreMesh(core_axis_name="c", subcore_axis_name="s")

    @functools.partial(
        pl.kernel,
        out_type=jax.ShapeDtypeStruct((NC, npad, 128), jnp.float32),
        mesh=mesh,
        scratch_types=[
            pltpu.VMEM((K,), jnp.int32),
            pltpu.VMEM((K,), jnp.int32),
            pltpu.VMEM((K, 128), jnp.float32),
            pltpu.VMEM_SHARED((npad, 128), jnp.float32),
            pltpu.SemaphoreType.DMA,
        ],
    )
    def msg_kernel(gs_hbm, src_hbm, dst_hbm, out_hbm,
                   src_v, dst_v, rows_v, acc, sem):
        c = lax.axis_index("c")
        s = lax.axis_index("s")
        wid = c * NS + s

        def init(j, carry):
            row = s * sp + j * K
            pltpu.sync_copy(gs_hbm.at[pl.ds(row, K)], rows_v)
            pltpu.sync_copy(rows_v, acc.at[pl.ds(row, K)])
            return carry

        lax.fori_loop(0, sp // K, init, 0)
        plsc.subcore_barrier()
        base0 = wid * ew

        def chunk(j, carry):
            base = base0 + j * K
            pltpu.sync_copy(src_hbm.at[pl.ds(base, K)], src_v)
            pltpu.sync_copy(dst_hbm.at[pl.ds(base, K)], dst_v)
            pltpu.async_copy(gs_hbm.at[src_v], rows_v, sem).wait()
            pltpu.sync_copy(rows_v, acc.at[dst_v], add=True)
            return carry

        lax.fori_loop(0, nch, chunk, 0)
        plsc.subcore_barrier()

        def readout(j, carry):
            row = s * sp + j * K
            pltpu.sync_copy(acc.at[pl.ds(row, K)], rows_v)
            pltpu.sync_copy(rows_v, out_hbm.at[c, pl.ds(row, K)])
            return carry

        lax.fori_loop(0, sp // K, readout, 0)

    return msg_kernel(gs_pad, src_pad, dst_pad)


def _dis_body(d_ref, o_ref):
    o_ref[...] = lax.rsqrt(d_ref[0] + d_ref[1] + 1.0)


def _tc_dis(deg_parts, n):
    npad = n + DR
    d3 = deg_parts.reshape(NC, npad // 128, 128)
    out = pl.pallas_call(
        _dis_body,
        out_shape=jax.ShapeDtypeStruct((npad // 128, 128), jnp.float32),
    )(d3)
    return out.reshape(npad)[:n].reshape(n, 1)


def _tc0_body(x_ref, dis_ref, w_ref, o_ref):
    o_ref[...] = jnp.dot(x_ref[...], w_ref[...],
                         preferred_element_type=jnp.float32) * dis_ref[...]


def _tc0(x, dis, w, n):
    d = x.shape[1]
    return pl.pallas_call(
        _tc0_body,
        grid=(n // BM,),
        in_specs=[
            pl.BlockSpec((BM, d), lambda i: (i, 0)),
            pl.BlockSpec((BM, 1), lambda i: (i, 0)),
            pl.BlockSpec((d, 128), lambda i: (0, 0)),
        ],
        out_specs=pl.BlockSpec((BM, 128), lambda i: (i, 0)),
        out_shape=jax.ShapeDtypeStruct((n + DR, 128), jnp.float32),
    )(x, dis, w)


def _combine_emb(p_ref, gs_ref, dis_ref, b_ref, g_ref, bt_ref):
    tot = p_ref[0] + p_ref[1] - gs_ref[...]
    h = tot * dis_ref[...] + b_ref[...]
    return jnp.maximum(h * (g_ref[...] * _BN_INV) + bt_ref[...], 0.0)


def _layer_body(p_ref, gs_ref, dis_ref, b_ref, g_ref, bt_ref, w_ref, o_ref):
    h = _combine_emb(p_ref, gs_ref, dis_ref, b_ref, g_ref, bt_ref)
    o_ref[...] = jnp.dot(h, w_ref[...],
                         preferred_element_type=jnp.float32) * dis_ref[...]


def _tc_layer(p, gs, dis, b, gamma, beta, w_next, n):
    vec = pl.BlockSpec((1, 128), lambda i: (0, 0))
    return pl.pallas_call(
        _layer_body,
        grid=(n // BM,),
        in_specs=[
            pl.BlockSpec((NC, BM, 128), lambda i: (0, i, 0)),
            pl.BlockSpec((BM, 128), lambda i: (i, 0)),
            pl.BlockSpec((BM, 1), lambda i: (i, 0)),
            vec, vec, vec,
            pl.BlockSpec((128, 128), lambda i: (0, 0)),
        ],
        out_specs=pl.BlockSpec((BM, 128), lambda i: (i, 0)),
        out_shape=jax.ShapeDtypeStruct((n + DR, 128), jnp.float32),
    )(p, gs, dis, b.reshape(1, 128), gamma.reshape(1, 128),
      beta.reshape(1, 128), w_next)


def _final_body(p_ref, gs_ref, dis_ref, batch_ref, b_ref, g_ref, bt_ref,
                w0_ref, b0_ref, w1_ref, b1_ref, w2_ref, b2_ref,
                np_ref, gsum_ref, gmax_ref, cnt_ref):
    i = pl.program_id(0)
    emb = _combine_emb(p_ref, gs_ref, dis_ref, b_ref, g_ref, bt_ref)
    z = jnp.maximum(jnp.dot(emb, w0_ref[...],
                            preferred_element_type=jnp.float32) + b0_ref[...], 0.0)
    z = jnp.maximum(jnp.dot(z, w1_ref[...],
                            preferred_element_type=jnp.float32) + b1_ref[...], 0.0)
    np_ref[...] = jnp.dot(z, w2_ref[...],
                          preferred_element_type=jnp.float32) + b2_ref[...]

    bt = batch_ref[...]
    onehot = (bt == lax.broadcasted_iota(jnp.int32, (1, 16), 1)
              ).astype(jnp.float32)

    @pl.when(i == 0)
    def _():
        gsum_ref[...] = jnp.zeros((16, 128), jnp.float32)
        cnt_ref[...] = jnp.zeros((16, 128), jnp.float32)
        gmax_ref[...] = jnp.full((16, 128), -jnp.inf, jnp.float32)

    gsum_ref[...] += lax.dot_general(onehot, emb, (((0,), (0,)), ((), ())),
                                     preferred_element_type=jnp.float32)
    cnt_ref[...] += jnp.broadcast_to(jnp.sum(onehot, axis=0)[:, None], (16, 128))
    for g in range(16):
        m = jnp.max(jnp.where(bt == g, emb, -jnp.inf), axis=0)
        gmax_ref[g, :] = jnp.maximum(gmax_ref[g, :], m)


def _tc_final(p, gs, dis, batch2, b, gamma, beta, nw0, nb0, nw1, nb1,
              nw2p, nb2p, n):
    vec = pl.BlockSpec((1, 128), lambda i: (0, 0))
    pool_spec = pl.BlockSpec((16, 128), lambda i: (0, 0))
    return pl.pallas_call(
        _final_body,
        grid=(n // BM,),
        in_specs=[
            pl.BlockSpec((NC, BM, 128), lambda i: (0, i, 0)),
            pl.BlockSpec((BM, 128), lambda i: (i, 0)),
            pl.BlockSpec((BM, 1), lambda i: (i, 0)),
            pl.BlockSpec((BM, 1), lambda i: (i, 0)),
            vec, vec, vec,
            pl.BlockSpec((128, 64), lambda i: (0, 0)),
            pl.BlockSpec((1, 64), lambda i: (0, 0)),
            pl.BlockSpec((64, 32), lambda i: (0, 0)),
            pl.BlockSpec((1, 32), lambda i: (0, 0)),
            pl.BlockSpec((32, 128), lambda i: (0, 0)),
            vec,
        ],
        out_specs=[
            pl.BlockSpec((BM, 128), lambda i: (i, 0)),
            pool_spec, pool_spec, pool_spec,
        ],
        out_shape=[
            jax.ShapeDtypeStruct((n, 128), jnp.float32),
            jax.ShapeDtypeStruct((16, 128), jnp.float32),
            jax.ShapeDtypeStruct((16, 128), jnp.float32),
            jax.ShapeDtypeStruct((16, 128), jnp.float32),
        ],
    )(p, gs, dis, batch2, b.reshape(1, 128), gamma.reshape(1, 128),
      beta.reshape(1, 128), nw0, nb0.reshape(1, 64), nw1, nb1.reshape(1, 32),
      nw2p, nb2p.reshape(1, 128))


def _graph_body(gsum_ref, gmax_ref, cnt_ref, w0a, w0b, b0, w1, b1, w2, b2,
                o_ref):
    gmean = gsum_ref[...] / jnp.maximum(cnt_ref[...], 1.0)
    g = jnp.maximum(
        jnp.dot(gmean, w0a[...], preferred_element_type=jnp.float32)
        + jnp.dot(gmax_ref[...], w0b[...], preferred_element_type=jnp.float32)
        + b0[...], 0.0)
    g = jnp.maximum(jnp.dot(g, w1[...],
                            preferred_element_type=jnp.float32) + b1[...], 0.0)
    o_ref[...] = jnp.dot(g, w2[...],
                         preferred_element_type=jnp.float32) + b2[...]


def _tc_graph(gsum, gmax, cnt, gw0a, gw0b, gb0, gw1, gb1, gw2p, gb2p):
    return pl.pallas_call(
        _graph_body,
        out_shape=jax.ShapeDtypeStruct((16, 128), jnp.float32),
    )(gsum, gmax, cnt, gw0a, gw0b, gb0.reshape(1, 128), gw1,
      gb1.reshape(1, 64), gw2p, gb2p.reshape(1, 128))


def kernel(x, edge_index, batch, params):
    n = x.shape[0]
    e = edge_index.shape[1]
    npad = n + DR
    ep = ((e + NW * K - 1) // (NW * K)) * (NW * K)
    pad = ep - e
    src = edge_index[0]
    dst = edge_index[1]
    if pad:
        src = jnp.concatenate([src, jnp.zeros((pad,), jnp.int32)])
        dump = n + (jnp.arange(pad, dtype=jnp.int32) % DR)
        dst = jnp.concatenate([dst, dump])

    deg_parts = _sc_degree(dst, n, ep)
    dis = _tc_dis(deg_parts, n)

    gs0 = _tc0(x, dis, params["W0"], n)
    p = _sc_message(gs0, src, dst, n, ep)
    gs1 = _tc_layer(p, gs0, dis, params["b0"], params["gamma0"],
                    params["beta0"], params["W1"], n)
    p = _sc_message(gs1, src, dst, n, ep)
    gs2 = _tc_layer(p, gs1, dis, params["b1"], params["gamma1"],
                    params["beta1"], params["W2"], n)
    p = _sc_message(gs2, src, dst, n, ep)

    nw2p = jnp.pad(params["nW2"], ((0, 0), (0, 126)))
    nb2p = jnp.pad(params["nb2"], (0, 126))
    node_pad, gsum, gmax, cnt = _tc_final(
        p, gs2, dis, batch.reshape(n, 1), params["b2"], params["gamma2"],
        params["beta2"], params["nW0"], params["nb0"], params["nW1"],
        params["nb1"], nw2p, nb2p, n)

    gw2p = jnp.pad(params["gW2"], ((0, 0), (0, 126)))
    gb2p = jnp.pad(params["gb2"], (0, 126))
    graph_pad = _tc_graph(gsum, gmax, cnt, params["gW0"][:128],
                          params["gW0"][128:], params["gb0"], params["gW1"],
                          params["gb1"], gw2p, gb2p)
    return node_pad[:, :2], graph_pad[:, :2]
